# Initial kernel scaffold; baseline (speedup 1.0000x reference)
import functools

import jax
import jax.numpy as jnp
from jax import lax
from jax.experimental import pallas as pl
from jax.experimental.pallas import tpu as pltpu
from jax.experimental.pallas import tpu_sc as plsc

N = 65536
DEG = 64
COLS = 64
L = 16
NC = 2
NS = 16
NW = NC * NS
ROWS_W = N // NW
R = 8
E = R * DEG
CHUNK = 128
NCH = E // CHUNK
NB = ROWS_W // R
NQ = COLS // L


def _body(inp_h, cols2_h, vals_h, bias_h, out_h,
          idx_b, vals_b, rows_b, out_b, bias_b, sem0, sem1):
  sems = (sem0, sem1)
  wid = lax.axis_index("s") * NC + lax.axis_index("c")
  row0 = wid * ROWS_W
  e0 = row0 * DEG
  cb0 = e0 // CHUNK

  pltpu.sync_copy(bias_h.at[pl.ds(row0, ROWS_W)], bias_b)

  def load_block(g, b):
    pltpu.sync_copy(cols2_h.at[pl.ds(cb0 + g * NCH, NCH)], idx_b.at[b])
    pltpu.sync_copy(vals_h.at[pl.ds(e0 + g * E, E)], vals_b.at[b])
    for k in range(NCH):
      pltpu.async_copy(inp_h.at[idx_b.at[b, k]],
                       rows_b.at[b, pl.ds(k * CHUNK, CHUNK)], sems[b])

  def wait_block(b):
    pltpu.make_async_copy(inp_h.at[pl.ds(0, E)], rows_b.at[b], sems[b]).wait()

  def compute_block(g, b):
    def row_body(r, carry):
      bias_s = bias_b[g * R + r]
      ebase = r * DEG
      acc = [jnp.full((L,), bias_s, jnp.float32) for _ in range(NQ)]
      for j in range(DEG):
        e = ebase + j
        v = vals_b[b, e]
        for q in range(NQ):
          acc[q] = acc[q] + v * rows_b[b, e, pl.ds(q * L, L)]
      for q in range(NQ):
        out_b[r, pl.ds(q * L, L)] = acc[q]
      return carry
    lax.fori_loop(0, R, row_body, 0)
    pltpu.sync_copy(out_b, out_h.at[pl.ds(row0 + g * R, R)])

  load_block(0, 0)

  def outer(t, carry):
    for b in (0, 1):
      g = 2 * t + b

      @pl.when(g + 1 < NB)
      def _():
        load_block(g + 1, 1 - b)

      wait_block(b)
      compute_block(g, b)
    return carry

  lax.fori_loop(0, NB // 2, outer, 0)


@jax.jit
def _run(inp, cols2, vals, bias):
  mesh = plsc.VectorSubcoreMesh(core_axis_name="c", subcore_axis_name="s",
                                num_cores=NC, num_subcores=NS)
  return pl.kernel(
      _body,
      out_type=jax.ShapeDtypeStruct((N, COLS), jnp.float32),
      mesh=mesh,
      scratch_types=[
          pltpu.VMEM((2, NCH, CHUNK), jnp.int32),
          pltpu.VMEM((2, E), jnp.float32),
          pltpu.VMEM((2, E, COLS), jnp.float32),
          pltpu.VMEM((R, COLS), jnp.float32),
          pltpu.VMEM((ROWS_W,), jnp.float32),
          pltpu.SemaphoreType.DMA,
          pltpu.SemaphoreType.DMA,
      ],
  )(inp, cols2, vals, bias)


def kernel(inp, rows, cols, vals, bias):
  del rows
  cols2 = cols.reshape(-1, CHUNK)
  return _run(inp, cols2, vals, bias)


# speedup vs baseline: 42.5710x; 42.5710x over previous
import functools

import jax
import jax.numpy as jnp
from jax import lax
from jax.experimental import pallas as pl
from jax.experimental.pallas import tpu as pltpu
from jax.experimental.pallas import tpu_sc as plsc

N = 65536
DEG = 64
COLS = 64
L = 16
NC = 2
NS = 16
NW = NC * NS
ROWS_W = N // NW
R = 8
E = R * DEG
CHUNK = 128
NCH = E // CHUNK
NB = ROWS_W // R
NQ = COLS // L


def _body(inp_h, cols_h, vals_h, bias_h, out_h,
          idx_b, vals_b, rows_b, out_b, bias_b, sem0, sem1):
  sems = (sem0, sem1)
  wid = lax.axis_index("s") * NC + lax.axis_index("c")
  row0 = wid * ROWS_W
  e0 = row0 * DEG

  pltpu.sync_copy(bias_h.at[pl.ds(row0, ROWS_W)], bias_b.at[pl.ds(0, ROWS_W)])

  def load_block(g, b):
    pltpu.sync_copy(cols_h.at[pl.ds(e0 + g * E, E)], idx_b.at[b])
    pltpu.sync_copy(vals_h.at[pl.ds(e0 + g * E, E)], vals_b.at[b])
    for k in range(NCH):
      pltpu.async_copy(inp_h.at[idx_b.at[b, pl.ds(k * CHUNK, CHUNK)]],
                       rows_b.at[b, pl.ds(k * CHUNK, CHUNK)], sems[b])

  def wait_block(b):
    pltpu.make_async_copy(inp_h.at[pl.ds(0, E)], rows_b.at[b], sems[b]).wait()

  def compute_block(g, b):
    def row_body(r, carry):
      bias_s = bias_b[pl.ds(g * R + r, L)][0]
      ebase = r * DEG
      acc = [jnp.full((L,), bias_s, jnp.float32) for _ in range(NQ)]
      for t in range(DEG // L):
        vv = vals_b[b, pl.ds(ebase + t * L, L)]
        for j in range(L):
          e = ebase + t * L + j
          v = vv[j]
          for q in range(NQ):
            acc[q] = acc[q] + v * rows_b[b, e, pl.ds(q * L, L)]
      for q in range(NQ):
        out_b[r, pl.ds(q * L, L)] = acc[q]
      return carry
    lax.fori_loop(0, R, row_body, 0)
    pltpu.sync_copy(out_b, out_h.at[pl.ds(row0 + g * R, R)])

  load_block(0, 0)

  def outer(t, carry):
    for b in (0, 1):
      g = 2 * t + b

      @pl.when(g + 1 < NB)
      def _():
        load_block(g + 1, 1 - b)

      wait_block(b)
      compute_block(g, b)
    return carry

  lax.fori_loop(0, NB // 2, outer, 0)


@jax.jit
def _run(inp, cols, vals, bias):
  mesh = plsc.VectorSubcoreMesh(core_axis_name="c", subcore_axis_name="s",
                                num_cores=NC, num_subcores=NS)
  return pl.kernel(
      _body,
      out_type=jax.ShapeDtypeStruct((N, COLS), jnp.float32),
      mesh=mesh,
      compiler_params=pltpu.CompilerParams(use_tc_tiling_on_sc=False),
      scratch_types=[
          pltpu.VMEM((2, E), jnp.int32),
          pltpu.VMEM((2, E), jnp.float32),
          pltpu.VMEM((2, E, COLS), jnp.float32),
          pltpu.VMEM((R, COLS), jnp.float32),
          pltpu.VMEM((ROWS_W + L,), jnp.float32),
          pltpu.SemaphoreType.DMA,
          pltpu.SemaphoreType.DMA,
      ],
  )(inp, cols, vals, bias)


def kernel(inp, rows, cols, vals, bias):
  del rows
  return _run(inp, cols, vals, bias)


# speedup vs baseline: 60.0473x; 1.4105x over previous
import functools

import jax
import jax.numpy as jnp
from jax import lax
from jax.experimental import pallas as pl
from jax.experimental.pallas import tpu as pltpu
from jax.experimental.pallas import tpu_sc as plsc

N = 65536
DEG = 64
COLS = 64
L = 16
NC = 2
NS = 16
NW = NC * NS
ROWS_W = N // NW
R = 8
E = R * DEG
CHUNK = 128
NCH = E // CHUNK
NB = ROWS_W // R
NQ = COLS // L


def _body(inp_h, cols_h, vals_h, bias_h, out_h,
          idx_b, vals_b, rows_b, out_b, bias_b,
          sem_g0, sem_g1, sem_i0, sem_i1, sem_i2, sem_i3,
          sem_v0, sem_v1, sem_v2, sem_v3, sem_o0, sem_o1):
  sem_g = (sem_g0, sem_g1)
  sem_i = (sem_i0, sem_i1, sem_i2, sem_i3)
  sem_v = (sem_v0, sem_v1, sem_v2, sem_v3)
  sem_o = (sem_o0, sem_o1)
  wid = lax.axis_index("s") * NC + lax.axis_index("c")
  row0 = wid * ROWS_W
  e0 = row0 * DEG

  pltpu.sync_copy(bias_h.at[pl.ds(row0, ROWS_W)], bias_b.at[pl.ds(0, ROWS_W)])

  def fire_meta(g, s4):
    pltpu.async_copy(cols_h.at[pl.ds(e0 + g * E, E)], idx_b.at[s4], sem_i[s4])
    pltpu.async_copy(vals_h.at[pl.ds(e0 + g * E, E)], vals_b.at[s4], sem_v[s4])

  def fire_gathers(g, s4, s2):
    pltpu.make_async_copy(cols_h.at[pl.ds(0, E)], idx_b.at[s4],
                          sem_i[s4]).wait()
    for k in range(NCH):
      pltpu.async_copy(inp_h.at[idx_b.at[s4, pl.ds(k * CHUNK, CHUNK)]],
                       rows_b.at[s2, pl.ds(k * CHUNK, CHUNK)], sem_g[s2])

  def compute_block(g, s4, s2):
    pltpu.make_async_copy(inp_h.at[pl.ds(0, E)], rows_b.at[s2],
                          sem_g[s2]).wait()
    pltpu.make_async_copy(vals_h.at[pl.ds(0, E)], vals_b.at[s4],
                          sem_v[s4]).wait()

    def row_body(r, carry):
      bias_s = bias_b[pl.ds(g * R + r, L)][0]
      ebase = r * DEG
      acc = [jnp.full((L,), bias_s, jnp.float32) for _ in range(NQ)]
      for t in range(DEG // L):
        vv = vals_b[s4, pl.ds(ebase + t * L, L)]
        for j in range(L):
          e = ebase + t * L + j
          v = vv[j]
          for q in range(NQ):
            acc[q] = acc[q] + v * rows_b[s2, e, pl.ds(q * L, L)]
      for q in range(NQ):
        out_b[s2, r, pl.ds(q * L, L)] = acc[q]
      return carry
    lax.fori_loop(0, R, row_body, 0)
    pltpu.async_copy(out_b.at[s2], out_h.at[pl.ds(row0 + g * R, R)],
                     sem_o[s2])

  fire_meta(0, 0)
  fire_meta(1, 1)
  fire_gathers(0, 0, 0)

  def outer(t, carry):
    for b in range(4):
      g = 4 * t + b
      s4 = b
      s2 = b % 2

      @pl.when(g + 2 < NB)
      def _():
        fire_meta(g + 2, (s4 + 2) % 4)

      @pl.when(g + 1 < NB)
      def _():
        fire_gathers(g + 1, (s4 + 1) % 4, 1 - s2)

      @pl.when(g >= 2)
      def _():
        pltpu.make_async_copy(out_b.at[s2], out_h.at[pl.ds(row0, R)],
                              sem_o[s2]).wait()

      compute_block(g, s4, s2)
    return carry

  lax.fori_loop(0, NB // 4, outer, 0)

  for s2 in range(2):
    pltpu.make_async_copy(out_b.at[s2], out_h.at[pl.ds(row0, R)],
                          sem_o[s2]).wait()


@jax.jit
def _run(inp, cols, vals, bias):
  mesh = plsc.VectorSubcoreMesh(core_axis_name="c", subcore_axis_name="s",
                                num_cores=NC, num_subcores=NS)
  return pl.kernel(
      _body,
      out_type=jax.ShapeDtypeStruct((N, COLS), jnp.float32),
      mesh=mesh,
      compiler_params=pltpu.CompilerParams(use_tc_tiling_on_sc=False),
      scratch_types=[
          pltpu.VMEM((4, E), jnp.int32),
          pltpu.VMEM((4, E), jnp.float32),
          pltpu.VMEM((2, E, COLS), jnp.float32),
          pltpu.VMEM((2, R, COLS), jnp.float32),
          pltpu.VMEM((ROWS_W + L,), jnp.float32),
      ] + [pltpu.SemaphoreType.DMA] * 12,
  )(inp, cols, vals, bias)


def kernel(inp, rows, cols, vals, bias):
  del rows
  return _run(inp, cols, vals, bias)
